# Initial kernel scaffold; baseline (speedup 1.0000x reference)
import jax
import jax.numpy as jnp
from jax.experimental import pallas as pl


def kernel(partname_indices, pos_values, uuid_values, uuid_embedding):
    raise NotImplementedError("write your pallas kernel here")


# speedup vs baseline: 5.6669x; 5.6669x over previous
import functools

import jax
import jax.numpy as jnp
from jax import lax
from jax.experimental import pallas as pl
from jax.experimental.pallas import tpu as pltpu
from jax.experimental.pallas import tpu_sc as plsc

N = 16384
K = 10
WIDTH = 64
B = N * K

NC = 2
NS = 16
NW = NC * NS
B_PER_W = B // NW
CH = 640
NCH = B_PER_W // CH


def _gather_kernel(table_hbm, idx_hbm, out_hbm, idx_v, buf0, buf1, sem0, sem1):
    wid = lax.axis_index("s") * NC + lax.axis_index("c")
    base = wid * B_PER_W
    pltpu.sync_copy(idx_hbm.at[pl.ds(base, B_PER_W)], idx_v)

    bufs = (buf0, buf1)
    sems = (sem0, sem1)

    def start(c):
        return pltpu.async_copy(
            table_hbm.at[idx_v.at[pl.ds(c * CH, CH)]], bufs[c % 2], sems[c % 2]
        )

    pending = start(0)
    for c in range(NCH):
        nxt = pending
        if c + 1 < NCH:
            pending = start(c + 1)
        nxt.wait()
        pltpu.sync_copy(bufs[c % 2], out_hbm.at[pl.ds(base + c * CH, CH)])


@jax.jit
def _lookup(uuid_values_flat, uuid_embedding):
    mesh = plsc.VectorSubcoreMesh(core_axis_name="c", subcore_axis_name="s")
    k = functools.partial(
        pl.kernel,
        mesh=mesh,
        out_type=jax.ShapeDtypeStruct((B, WIDTH), jnp.float32),
        scratch_types=[
            pltpu.VMEM((B_PER_W,), jnp.int32),
            pltpu.VMEM((CH, WIDTH), jnp.float32),
            pltpu.VMEM((CH, WIDTH), jnp.float32),
            pltpu.SemaphoreType.DMA,
            pltpu.SemaphoreType.DMA,
        ],
        compiler_params=pltpu.CompilerParams(use_tc_tiling_on_sc=False),
    )(_gather_kernel)
    return k(uuid_embedding, uuid_values_flat)


def kernel(partname_indices, pos_values, uuid_values, uuid_embedding):
    flat = _lookup(uuid_values.reshape(-1), uuid_embedding)
    return flat.reshape(N, K * WIDTH)


# speedup vs baseline: 5.7158x; 1.0086x over previous
import functools

import jax
import jax.numpy as jnp
from jax import lax
from jax.experimental import pallas as pl
from jax.experimental.pallas import tpu as pltpu
from jax.experimental.pallas import tpu_sc as plsc

N = 16384
K = 10
WIDTH = 64
B = N * K

NC = 2
NS = 16
NW = NC * NS
B_PER_W = B // NW
CH = 320
NCH = B_PER_W // CH
NBUF = 4


def _gather_kernel(table_hbm, idx_hbm, out_hbm, idx_v, bufs, gsems, wsems):
    wid = lax.axis_index("s") * NC + lax.axis_index("c")
    base = wid * B_PER_W
    pltpu.sync_copy(idx_hbm.at[pl.ds(base, B_PER_W)], idx_v)

    def start_gather(c):
        b = c % NBUF
        return pltpu.async_copy(
            table_hbm.at[idx_v.at[pl.ds(c * CH, CH)]], bufs[b], gsems[b]
        )

    def start_write(c):
        b = c % NBUF
        return pltpu.async_copy(
            bufs[b], out_hbm.at[pl.ds(base + c * CH, CH)], wsems[b]
        )

    ghandles = [None] * NBUF
    whandles = [None] * NBUF
    for c in range(NCH + NBUF - 1):
        if c < NCH:
            b = c % NBUF
            if whandles[b] is not None:
                whandles[b].wait()
            ghandles[b] = start_gather(c)
        d = c - (NBUF - 1)
        if d >= 0:
            db = d % NBUF
            ghandles[db].wait()
            whandles[db] = start_write(d)
    for b in range(NBUF):
        if whandles[b] is not None:
            whandles[b].wait()


@jax.jit
def _lookup(uuid_values_flat, uuid_embedding):
    mesh = plsc.VectorSubcoreMesh(core_axis_name="c", subcore_axis_name="s")
    k = functools.partial(
        pl.kernel,
        mesh=mesh,
        out_type=jax.ShapeDtypeStruct((B, WIDTH), jnp.float32),
        scratch_types=[
            pltpu.VMEM((B_PER_W,), jnp.int32),
            [pltpu.VMEM((CH, WIDTH), jnp.float32) for _ in range(NBUF)],
            [pltpu.SemaphoreType.DMA for _ in range(NBUF)],
            [pltpu.SemaphoreType.DMA for _ in range(NBUF)],
        ],
        compiler_params=pltpu.CompilerParams(use_tc_tiling_on_sc=False),
    )(_gather_kernel)
    return k(uuid_embedding, uuid_values_flat)


def kernel(partname_indices, pos_values, uuid_values, uuid_embedding):
    flat = _lookup(uuid_values.reshape(-1), uuid_embedding)
    return flat.reshape(N, K * WIDTH)
